# Initial kernel scaffold; baseline (speedup 1.0000x reference)
import jax
import jax.numpy as jnp
from jax.experimental import pallas as pl


def kernel(src, colors, feats):
    raise NotImplementedError("write your pallas kernel here")


# speedup vs baseline: 1.4366x; 1.4366x over previous
import jax
import jax.numpy as jnp
from jax import lax
from jax.experimental import pallas as pl
from jax.experimental.pallas import tpu as pltpu

B, H, W = 2, 224, 224
K, D = 19, 1024
P = H * W
CHUNK = 1792
KPAD = 32


def _body(src_ref, colors_ref, table_ref, out_ref):
    q = (src_ref[0] * 127.5 + 127.5).astype(jnp.int32)
    match = None
    for c in range(3):
        eq = q[c:c + 1, :] == colors_ref[:, c:c + 1]
        match = eq if match is None else (match & eq)
    kvec = lax.broadcasted_iota(jnp.int32, (K, CHUNK), 0)
    cls = jnp.min(jnp.where(match, kvec, KPAD - 1), axis=0, keepdims=True)
    onehot = (cls == lax.broadcasted_iota(jnp.int32, (KPAD, CHUNK), 0))
    out_ref[0] = lax.dot_general(
        table_ref[...], onehot.astype(jnp.float32),
        (((1,), (0,)), ((), ())), preferred_element_type=jnp.float32)


def kernel(src, colors, feats):
    src_flat = src.reshape(B, 3, P)
    colors_i = colors.astype(jnp.int32)
    table = jnp.zeros((D, KPAD), jnp.float32).at[:, :K].set(feats.T)
    out = pl.pallas_call(
        _body,
        grid=(B, P // CHUNK),
        in_specs=[
            pl.BlockSpec((1, 3, CHUNK), lambda b, j: (b, 0, j)),
            pl.BlockSpec((K, 3), lambda b, j: (0, 0)),
            pl.BlockSpec((D, KPAD), lambda b, j: (0, 0)),
        ],
        out_specs=pl.BlockSpec((1, D, CHUNK), lambda b, j: (b, 0, j)),
        out_shape=jax.ShapeDtypeStruct((B, D, P), jnp.float32),
        compiler_params=pltpu.CompilerParams(
            dimension_semantics=("parallel", "parallel")),
    )(src_flat, colors_i, table)
    return out.reshape(B, D, H, W)
